# baseline (speedup 1.0000x reference)
import functools

import jax
import jax.numpy as jnp
from jax.experimental import pallas as pl
from jax.experimental.pallas import tpu as pltpu

M = 2 * 77
K = 49408
N = 768
KB = 4096
NB = 384
NSTEPS = -(-K // KB)


def _body(a_ref, b_ref, p_ref, o_ref):
    k = pl.program_id(1)

    def full_dot():
        return jnp.dot(a_ref[...].astype(jnp.bfloat16),
                       b_ref[...].astype(jnp.bfloat16),
                       preferred_element_type=jnp.float32)

    def masked_dot():
        valid = K - (NSTEPS - 1) * KB
        a = a_ref[...]
        b = b_ref[...]
        a = jnp.where(
            jax.lax.broadcasted_iota(jnp.int32, a.shape, 1) < valid, a, 0.0)
        b = jnp.where(
            jax.lax.broadcasted_iota(jnp.int32, b.shape, 0) < valid, b, 0.0)
        return jnp.dot(a.astype(jnp.bfloat16), b.astype(jnp.bfloat16),
                       preferred_element_type=jnp.float32)

    partial = jax.lax.cond(k == NSTEPS - 1, masked_dot, full_dot)

    @pl.when(k == 0)
    def _init():
        p = p_ref[...]
        o_ref[...] = partial + jnp.concatenate([p, p], axis=0)

    @pl.when(k > 0)
    def _acc():
        o_ref[...] += partial


@jax.jit
def kernel(input_ids, token_weight, position_weight):
    batch, seq, _ = input_ids.shape
    a2d = input_ids.reshape(batch * seq, K)
    out2d = pl.pallas_call(
        _body,
        grid=(N // NB, NSTEPS),
        in_specs=[
            pl.BlockSpec((M, KB), lambda n, k: (0, k)),
            pl.BlockSpec((KB, NB), lambda n, k: (k, n)),
            pl.BlockSpec((seq, NB), lambda n, k: (0, n)),
        ],
        out_specs=pl.BlockSpec((M, NB), lambda n, k: (0, n)),
        out_shape=jax.ShapeDtypeStruct((M, N), jnp.float32),
        compiler_params=pltpu.CompilerParams(
            dimension_semantics=("parallel", "arbitrary")),
    )(a2d, token_weight, position_weight)
    return out2d.reshape(batch, seq, N)


# speedup vs baseline: 1.0651x; 1.0651x over previous
import jax
import jax.numpy as jnp
from jax.experimental import pallas as pl
from jax.experimental.pallas import tpu as pltpu

M = 2 * 77
K = 49408
N = 768
KB = 1024
NSTREAMS = 4
SLAB = NSTREAMS * KB
NSTEPS = -(-K // SLAB)
NBLK = -(-K // KB) - 1


def _body(a_ref, b0_ref, b1_ref, b2_ref, b3_ref, p_ref, o_ref):
    k = pl.program_id(0)
    b_refs = [b0_ref, b1_ref, b2_ref, b3_ref]

    def full_slab():
        acc = jnp.zeros((M, N), jnp.float32)
        for j in range(NSTREAMS):
            a = a_ref[:, j * KB:(j + 1) * KB].astype(jnp.bfloat16)
            acc += jnp.dot(a, b_refs[j][...].astype(jnp.bfloat16),
                           preferred_element_type=jnp.float32)
        return acc

    def tail_slab():
        valid = K - (NSTEPS - 1) * SLAB
        a = a_ref[:, :KB]
        b = b0_ref[...]
        a = jnp.where(
            jax.lax.broadcasted_iota(jnp.int32, a.shape, 1) < valid, a, 0.0)
        b = jnp.where(
            jax.lax.broadcasted_iota(jnp.int32, b.shape, 0) < valid, b, 0.0)
        return jnp.dot(a.astype(jnp.bfloat16), b.astype(jnp.bfloat16),
                       preferred_element_type=jnp.float32)

    partial = jax.lax.cond(k == NSTEPS - 1, tail_slab, full_slab)

    @pl.when(k == 0)
    def _init():
        p = p_ref[...]
        o_ref[...] = partial + jnp.concatenate([p, p], axis=0)

    @pl.when(k > 0)
    def _acc():
        o_ref[...] += partial


def _b_spec(j):
    return pl.BlockSpec(
        (KB, N), lambda k, j=j: (jnp.minimum(NSTREAMS * k + j, NBLK), 0))


@jax.jit
def kernel(input_ids, token_weight, position_weight):
    batch, seq, _ = input_ids.shape
    a2d = input_ids.reshape(batch * seq, K)
    out2d = pl.pallas_call(
        _body,
        grid=(NSTEPS,),
        in_specs=[
            pl.BlockSpec((M, SLAB), lambda k: (0, k)),
            _b_spec(0), _b_spec(1), _b_spec(2), _b_spec(3),
            pl.BlockSpec((seq, N), lambda k: (0, 0)),
        ],
        out_specs=pl.BlockSpec((M, N), lambda k: (0, 0)),
        out_shape=jax.ShapeDtypeStruct((M, N), jnp.float32),
        compiler_params=pltpu.CompilerParams(
            dimension_semantics=("arbitrary",)),
    )(a2d, token_weight, token_weight, token_weight, token_weight,
      position_weight)
    return out2d.reshape(batch, seq, N)


# speedup vs baseline: 1.7577x; 1.6502x over previous
import jax
import jax.numpy as jnp
from jax.experimental import pallas as pl
from jax.experimental.pallas import tpu as pltpu

B = 2
S = 77
K = 49408
N = 768
KB = 4096
NSTEPS = -(-K // KB)


def _body(a_ref, b_ref, p_ref, o_ref):
    k = pl.program_id(0)

    def full_dot():
        bm = b_ref[...].astype(jnp.bfloat16)
        return tuple(
            jnp.dot(a_ref[i].astype(jnp.bfloat16), bm,
                    preferred_element_type=jnp.float32)
            for i in range(B))

    def tail_dot():
        valid = K - (NSTEPS - 1) * KB
        bm = b_ref[...]
        bm = jnp.where(
            jax.lax.broadcasted_iota(jnp.int32, bm.shape, 0) < valid, bm, 0.0
        ).astype(jnp.bfloat16)
        outs = []
        for i in range(B):
            a = a_ref[i]
            a = jnp.where(
                jax.lax.broadcasted_iota(jnp.int32, a.shape, 1) < valid,
                a, 0.0).astype(jnp.bfloat16)
            outs.append(jnp.dot(a, bm, preferred_element_type=jnp.float32))
        return tuple(outs)

    partials = jax.lax.cond(k == NSTEPS - 1, tail_dot, full_dot)

    @pl.when(k == 0)
    def _init():
        p = p_ref[...]
        for i in range(B):
            o_ref[i] = partials[i] + p

    @pl.when(k > 0)
    def _acc():
        for i in range(B):
            o_ref[i] += partials[i]


@jax.jit
def kernel(input_ids, token_weight, position_weight):
    return pl.pallas_call(
        _body,
        grid=(NSTEPS,),
        in_specs=[
            pl.BlockSpec((B, S, KB), lambda k: (0, 0, k)),
            pl.BlockSpec((KB, N), lambda k: (k, 0)),
            pl.BlockSpec((S, N), lambda k: (0, 0)),
        ],
        out_specs=pl.BlockSpec((B, S, N), lambda k: (0, 0, 0)),
        out_shape=jax.ShapeDtypeStruct((B, S, N), jnp.float32),
        compiler_params=pltpu.CompilerParams(
            dimension_semantics=("arbitrary",)),
    )(input_ids, token_weight, position_weight)


# speedup vs baseline: 1.7633x; 1.0032x over previous
import jax
import jax.numpy as jnp
from jax.experimental import pallas as pl
from jax.experimental.pallas import tpu as pltpu

B = 2
S = 77
K = 49408
N = 768
KB = 1024
NSTREAMS = 4
SLAB = NSTREAMS * KB
NSTEPS = -(-K // SLAB)
NBLK = K // KB


def _body(a_ref, b0_ref, b1_ref, b2_ref, b3_ref, p_ref, o_ref):
    k = pl.program_id(0)
    b_refs = (b0_ref, b1_ref, b2_ref, b3_ref)

    def full_slab():
        accs = [jnp.zeros((S, N), jnp.float32) for _ in range(B)]
        for j in range(NSTREAMS):
            bm = b_refs[j][...].astype(jnp.bfloat16)
            for i in range(B):
                a = a_ref[i, :, j * KB:(j + 1) * KB].astype(jnp.bfloat16)
                accs[i] += jnp.dot(a, bm, preferred_element_type=jnp.float32)
        return tuple(accs)

    def tail_slab():
        valid = K - (NSTEPS - 1) * SLAB
        bm = b0_ref[...]
        bm = jnp.where(
            jax.lax.broadcasted_iota(jnp.int32, bm.shape, 0) < valid, bm, 0.0
        ).astype(jnp.bfloat16)
        outs = []
        for i in range(B):
            a = a_ref[i, :, :KB]
            a = jnp.where(
                jax.lax.broadcasted_iota(jnp.int32, a.shape, 1) < valid,
                a, 0.0).astype(jnp.bfloat16)
            outs.append(jnp.dot(a, bm, preferred_element_type=jnp.float32))
        return tuple(outs)

    partials = jax.lax.cond(k == NSTEPS - 1, tail_slab, full_slab)

    @pl.when(k == 0)
    def _init():
        p = p_ref[...]
        for i in range(B):
            o_ref[i] = partials[i] + p

    @pl.when(k > 0)
    def _acc():
        for i in range(B):
            o_ref[i] += partials[i]


def _b_spec(j):
    return pl.BlockSpec(
        (KB, N), lambda k, j=j: (jnp.minimum(NSTREAMS * k + j, NBLK), 0))


@jax.jit
def kernel(input_ids, token_weight, position_weight):
    return pl.pallas_call(
        _body,
        grid=(NSTEPS,),
        in_specs=[
            pl.BlockSpec((B, S, SLAB), lambda k: (0, 0, k)),
            _b_spec(0), _b_spec(1), _b_spec(2), _b_spec(3),
            pl.BlockSpec((S, N), lambda k: (0, 0)),
        ],
        out_specs=pl.BlockSpec((B, S, N), lambda k: (0, 0, 0)),
        out_shape=jax.ShapeDtypeStruct((B, S, N), jnp.float32),
        compiler_params=pltpu.CompilerParams(
            dimension_semantics=("arbitrary",)),
    )(input_ids, token_weight, token_weight, token_weight, token_weight,
      position_weight)


# speedup vs baseline: 2.5390x; 1.4399x over previous
import jax
import jax.numpy as jnp
from jax.experimental import pallas as pl
from jax.experimental.pallas import tpu as pltpu

B = 2
S = 77
K = 49408
N = 768
KB = 4096
NSTEPS = -(-K // KB)


def _body(a_ref, b_ref, p_ref, o_ref):
    k = pl.program_id(0)

    def full_dot():
        a = a_ref[...].reshape(S * B, KB)
        return jnp.dot(a.astype(jnp.bfloat16),
                       b_ref[...].astype(jnp.bfloat16),
                       preferred_element_type=jnp.float32)

    def tail_dot():
        valid = K - (NSTEPS - 1) * KB
        a = a_ref[...].reshape(S * B, KB)
        a = jnp.where(
            jax.lax.broadcasted_iota(jnp.int32, a.shape, 1) < valid, a, 0.0)
        bm = b_ref[...]
        bm = jnp.where(
            jax.lax.broadcasted_iota(jnp.int32, bm.shape, 0) < valid, bm, 0.0)
        return jnp.dot(a.astype(jnp.bfloat16), bm.astype(jnp.bfloat16),
                       preferred_element_type=jnp.float32)

    partial = jax.lax.cond(k == NSTEPS - 1, tail_dot, full_dot)

    @pl.when(k == 0)
    def _init():
        p = jnp.broadcast_to(p_ref[...][:, None, :], (S, B, N))
        o_ref[...] = partial.reshape(S, B, N) + p

    @pl.when(k > 0)
    def _acc():
        o_ref[...] += partial.reshape(S, B, N)


@jax.jit
def kernel(input_ids, token_weight, position_weight):
    a_t = jnp.transpose(input_ids, (1, 0, 2))
    out_t = pl.pallas_call(
        _body,
        grid=(NSTEPS,),
        in_specs=[
            pl.BlockSpec((S, B, KB), lambda k: (0, 0, k)),
            pl.BlockSpec((KB, N), lambda k: (k, 0)),
            pl.BlockSpec((S, N), lambda k: (0, 0)),
        ],
        out_specs=pl.BlockSpec((S, B, N), lambda k: (0, 0, 0)),
        out_shape=jax.ShapeDtypeStruct((S, B, N), jnp.float32),
        compiler_params=pltpu.CompilerParams(
            dimension_semantics=("arbitrary",)),
    )(a_t, token_weight, position_weight)
    return jnp.transpose(out_t, (1, 0, 2))
